# baseline (speedup 1.0000x reference)
import functools

import jax
import jax.numpy as jnp
from jax import lax
from jax.experimental import pallas as pl
from jax.experimental.pallas import tpu as pltpu
from jax.experimental.pallas import tpu_sc as plsc

NC = 2
NS = 16
NW = NC * NS
LANES = 16


def _emb_body(batch, seq, s_per_w, rows_chunk, d,
              ids_hbm, tok_hbm, pos_hbm, out_hbm,
              idx_v, tok_v, pos_v,
              gsem0, gsem1, gsem2, ssem0, ssem1, ssem2, psem):
    cpb = s_per_w // rows_chunk
    n_chunks = batch * cpb
    wid = lax.axis_index("s") * NC + lax.axis_index("c")
    seq0 = wid * s_per_w

    gsems = (gsem0, gsem1, gsem2)
    ssems = (ssem0, ssem1, ssem2)
    nbuf = 3

    pos_fetch = pltpu.async_copy(pos_hbm.at[pl.ds(seq0, s_per_w)], pos_v, psem)
    for b in range(batch):
        pltpu.sync_copy(
            ids_hbm.at[pl.ds(b * seq + seq0, s_per_w)], idx_v.at[b])

    def chunk_coords(k):
        b, h = divmod(k, cpb)
        row0 = b * seq + seq0 + h * rows_chunk
        return b, h, row0

    def start_gather(k):
        b, h, _ = chunk_coords(k)
        buf = k % nbuf
        return pltpu.async_copy(
            tok_hbm.at[idx_v.at[b, pl.ds(h * rows_chunk, rows_chunk)]],
            tok_v.at[buf], gsems[buf])

    fetches = {0: start_gather(0), 1: start_gather(1)}
    stores = {}
    for k in range(n_chunks):
        buf = k % nbuf
        _, h, row0 = chunk_coords(k)
        if k >= 1:
            stores.pop(k - 1).wait()
        if k + 2 < n_chunks:
            fetches[k + 2] = start_gather(k + 2)
        fetches.pop(k).wait()
        if k == 0:
            pos_fetch.wait()

        @plsc.parallel_loop(0, rows_chunk, unroll=4)
        def row_body(r):
            for c in range(d // LANES):
                sl = pl.ds(c * LANES, LANES)
                plsc.addupdate(tok_v.at[buf, r, sl],
                               pos_v[h * rows_chunk + r, sl])
        stores[k] = pltpu.async_copy(
            tok_v.at[buf], out_hbm.at[pl.ds(row0, rows_chunk)], ssems[buf])
    stores.pop(n_chunks - 1).wait()


def kernel(input_ids, token_table, pos_table):
    batch, seq = input_ids.shape
    vocab, d = token_table.shape
    n = batch * seq
    ids_flat = input_ids.reshape(n).astype(jnp.int32)

    s_per_w = seq // NW
    rows_chunk = 32

    mesh = plsc.VectorSubcoreMesh(core_axis_name="c", subcore_axis_name="s")

    run = functools.partial(
        pl.kernel,
        mesh=mesh,
        out_type=jax.ShapeDtypeStruct((n, d), jnp.float32),
        scratch_types=[
            pltpu.VMEM((batch, s_per_w), jnp.int32),
            pltpu.VMEM((3, rows_chunk, d), jnp.float32),
            pltpu.VMEM((s_per_w, d), jnp.float32),
            pltpu.SemaphoreType.DMA,
            pltpu.SemaphoreType.DMA,
            pltpu.SemaphoreType.DMA,
            pltpu.SemaphoreType.DMA,
            pltpu.SemaphoreType.DMA,
            pltpu.SemaphoreType.DMA,
            pltpu.SemaphoreType.DMA,
        ],
    )(functools.partial(_emb_body, batch, seq, s_per_w, rows_chunk, d))

    out = run(ids_flat, token_table, pos_table)
    return out.reshape(batch, seq, d)


# speedup vs baseline: 1.0136x; 1.0136x over previous
import functools

import jax
import jax.numpy as jnp
from jax import lax
from jax.experimental import pallas as pl
from jax.experimental.pallas import tpu as pltpu
from jax.experimental.pallas import tpu_sc as plsc

NC = 2
NS = 16
NW = NC * NS
LANES = 16


def _emb_body(batch, seq, s_per_w, rows_chunk, d,
              ids_hbm, tok_hbm, pos_hbm, out_hbm,
              idx_v, tok_v, pos_v,
              gsem0, gsem1, gsem2, ssem0, ssem1, ssem2, psem):
    cpb = s_per_w // rows_chunk
    n_chunks = batch * cpb
    wid = lax.axis_index("s") * NC + lax.axis_index("c")
    seq0 = wid * s_per_w

    gsems = (gsem0, gsem1, gsem2)
    ssems = (ssem0, ssem1, ssem2)
    nbuf = 2

    pos_fetch = pltpu.async_copy(pos_hbm.at[pl.ds(seq0, s_per_w)], pos_v, psem)
    for b in range(batch):
        pltpu.sync_copy(
            ids_hbm.at[pl.ds(b * seq + seq0, s_per_w)], idx_v.at[b])

    def chunk_coords(k):
        b, h = divmod(k, cpb)
        row0 = b * seq + seq0 + h * rows_chunk
        return b, h, row0

    def start_gather(k):
        b, h, _ = chunk_coords(k)
        buf = k % nbuf
        return pltpu.async_copy(
            tok_hbm.at[idx_v.at[b, pl.ds(h * rows_chunk, rows_chunk)]],
            tok_v.at[buf], gsems[buf])

    fetches = {0: start_gather(0)}
    stores = {}
    for k in range(n_chunks):
        buf = k % nbuf
        _, h, row0 = chunk_coords(k)
        if k >= 1:
            stores.pop(k - 1).wait()
        if k + 1 < n_chunks:
            fetches[k + 1] = start_gather(k + 1)
        fetches.pop(k).wait()
        if k == 0:
            pos_fetch.wait()

        @plsc.parallel_loop(0, rows_chunk, unroll=4)
        def row_body(r):
            for c in range(d // LANES):
                sl = pl.ds(c * LANES, LANES)
                plsc.addupdate(tok_v.at[buf, r, sl],
                               pos_v[h * rows_chunk + r, sl])
        stores[k] = pltpu.async_copy(
            tok_v.at[buf], out_hbm.at[pl.ds(row0, rows_chunk)], ssems[buf])
    stores.pop(n_chunks - 1).wait()


def kernel(input_ids, token_table, pos_table):
    batch, seq = input_ids.shape
    vocab, d = token_table.shape
    n = batch * seq
    ids_flat = input_ids.reshape(n).astype(jnp.int32)

    s_per_w = seq // NW
    rows_chunk = 32

    mesh = plsc.VectorSubcoreMesh(core_axis_name="c", subcore_axis_name="s")

    run = functools.partial(
        pl.kernel,
        mesh=mesh,
        out_type=jax.ShapeDtypeStruct((n, d), jnp.float32),
        scratch_types=[
            pltpu.VMEM((batch, s_per_w), jnp.int32),
            pltpu.VMEM((2, rows_chunk, d), jnp.float32),
            pltpu.VMEM((s_per_w, d), jnp.float32),
            pltpu.SemaphoreType.DMA,
            pltpu.SemaphoreType.DMA,
            pltpu.SemaphoreType.DMA,
            pltpu.SemaphoreType.DMA,
            pltpu.SemaphoreType.DMA,
            pltpu.SemaphoreType.DMA,
            pltpu.SemaphoreType.DMA,
        ],
    )(functools.partial(_emb_body, batch, seq, s_per_w, rows_chunk, d))

    out = run(ids_flat, token_table, pos_table)
    return out.reshape(batch, seq, d)


# speedup vs baseline: 1.1119x; 1.0970x over previous
import functools

import jax
import jax.numpy as jnp
from jax import lax
from jax.experimental import pallas as pl
from jax.experimental.pallas import tpu as pltpu
from jax.experimental.pallas import tpu_sc as plsc

NC = 2
NS = 16
NW = NC * NS
LANES = 16


def _emb_body(batch, seq, s_per_w, rows_chunk, d,
              ids_hbm, tok_hbm, pos_hbm, out_hbm,
              idx_v, tok_v, pos_v,
              gsem0, gsem1, gsem2, ssem0, ssem1, ssem2, psem):
    cpb = s_per_w // rows_chunk
    n_chunks = batch * cpb
    wid = lax.axis_index("s") * NC + lax.axis_index("c")
    seq0 = wid * s_per_w

    gsems = (gsem0, gsem1, gsem2)
    ssems = (ssem0, ssem1, ssem2)
    nbuf = 2

    pos_fetch = pltpu.async_copy(pos_hbm.at[pl.ds(seq0, s_per_w)], pos_v, psem)
    for b in range(batch):
        pltpu.sync_copy(
            ids_hbm.at[pl.ds(b * seq + seq0, s_per_w)], idx_v.at[b])

    def chunk_coords(k):
        b, h = divmod(k, cpb)
        row0 = b * seq + seq0 + h * rows_chunk
        return b, h, row0

    def start_gather(k):
        b, h, _ = chunk_coords(k)
        buf = k % nbuf
        return pltpu.async_copy(
            tok_hbm.at[idx_v.at[b, pl.ds(h * rows_chunk, rows_chunk)]],
            tok_v.at[buf], gsems[buf])

    fetches = {0: start_gather(0)}
    stores = {}
    for k in range(n_chunks):
        buf = k % nbuf
        _, h, row0 = chunk_coords(k)
        if k >= 1:
            stores.pop(k - 1).wait()
        if k + 1 < n_chunks:
            fetches[k + 1] = start_gather(k + 1)
        fetches.pop(k).wait()
        if k == 0:
            pos_fetch.wait()

        @plsc.parallel_loop(0, rows_chunk, unroll=2)
        def row_body(r):
            for c in range(d // LANES):
                sl = pl.ds(c * LANES, LANES)
                plsc.addupdate(tok_v.at[buf, r, sl],
                               pos_v[h * rows_chunk + r, sl])
        stores[k] = pltpu.async_copy(
            tok_v.at[buf], out_hbm.at[pl.ds(row0, rows_chunk)], ssems[buf])
    stores.pop(n_chunks - 1).wait()


def kernel(input_ids, token_table, pos_table):
    batch, seq = input_ids.shape
    vocab, d = token_table.shape
    n = batch * seq
    ids_flat = input_ids.reshape(n).astype(jnp.int32)

    s_per_w = seq // NW
    rows_chunk = 32

    mesh = plsc.VectorSubcoreMesh(core_axis_name="c", subcore_axis_name="s")

    run = functools.partial(
        pl.kernel,
        mesh=mesh,
        out_type=jax.ShapeDtypeStruct((n, d), jnp.float32),
        scratch_types=[
            pltpu.VMEM((batch, s_per_w), jnp.int32),
            pltpu.VMEM((2, rows_chunk, d), jnp.float32),
            pltpu.VMEM((s_per_w, d), jnp.float32),
            pltpu.SemaphoreType.DMA,
            pltpu.SemaphoreType.DMA,
            pltpu.SemaphoreType.DMA,
            pltpu.SemaphoreType.DMA,
            pltpu.SemaphoreType.DMA,
            pltpu.SemaphoreType.DMA,
            pltpu.SemaphoreType.DMA,
        ],
    )(functools.partial(_emb_body, batch, seq, s_per_w, rows_chunk, d))

    out = run(ids_flat, token_table, pos_table)
    return out.reshape(batch, seq, d)


# speedup vs baseline: 1.2328x; 1.1087x over previous
import functools

import jax
import jax.numpy as jnp
from jax import lax
from jax.experimental import pallas as pl
from jax.experimental.pallas import tpu as pltpu
from jax.experimental.pallas import tpu_sc as plsc

NC = 2
NS = 16
NW = NC * NS
LANES = 16


def _emb_body(batch, seq, s_per_w, rows_wave, d,
              ids_hbm, tok_hbm, pos_hbm, out_hbm,
              idx_v, tok_v, pos_v, *sems):
    n_waves = s_per_w // rows_wave
    gsems = (sems[:batch], sems[batch:2 * batch])
    ssems = (sems[2 * batch:3 * batch], sems[3 * batch:4 * batch])
    psems = sems[4 * batch:4 * batch + 2]
    wid = lax.axis_index("s") * NC + lax.axis_index("c")
    seq0 = wid * s_per_w

    for b in range(batch):
        pltpu.sync_copy(
            ids_hbm.at[pl.ds(b * seq + seq0, s_per_w)], idx_v.at[b])

    def start_wave(w):
        wb = w % 2
        h0 = w * rows_wave
        fetches = [pltpu.async_copy(
            pos_hbm.at[pl.ds(seq0 + h0, rows_wave)], pos_v.at[wb], psems[wb])]
        for b in range(batch):
            fetches.append(pltpu.async_copy(
                tok_hbm.at[idx_v.at[b, pl.ds(h0, rows_wave)]],
                tok_v.at[wb, b], gsems[wb][b]))
        return fetches

    fetches = {0: start_wave(0)}
    stores = {}
    for w in range(n_waves):
        wb = w % 2
        h0 = w * rows_wave
        if w >= 1:
            for st in stores.pop(w - 1):
                st.wait()
        if w + 1 < n_waves:
            fetches[w + 1] = start_wave(w + 1)
        for f in fetches.pop(w):
            f.wait()

        @plsc.parallel_loop(0, rows_wave, unroll=1)
        def p_body(p):
            for c in range(d // LANES):
                sl = pl.ds(c * LANES, LANES)
                x = pos_v[wb, p, sl]
                for b in range(batch):
                    plsc.addupdate(tok_v.at[wb, b, p, sl], x)

        stores[w] = [pltpu.async_copy(
            tok_v.at[wb, b],
            out_hbm.at[pl.ds(b * seq + seq0 + h0, rows_wave)], ssems[wb][b])
            for b in range(batch)]
    for w in sorted(stores):
        for st in stores[w]:
            st.wait()


def kernel(input_ids, token_table, pos_table):
    batch, seq = input_ids.shape
    vocab, d = token_table.shape
    n = batch * seq
    ids_flat = input_ids.reshape(n).astype(jnp.int32)

    s_per_w = seq // NW
    rows_wave = 16

    mesh = plsc.VectorSubcoreMesh(core_axis_name="c", subcore_axis_name="s")

    run = functools.partial(
        pl.kernel,
        mesh=mesh,
        out_type=jax.ShapeDtypeStruct((n, d), jnp.float32),
        scratch_types=[
            pltpu.VMEM((batch, s_per_w), jnp.int32),
            pltpu.VMEM((2, batch, rows_wave, d), jnp.float32),
            pltpu.VMEM((2, rows_wave, d), jnp.float32),
        ] + [pltpu.SemaphoreType.DMA] * (4 * batch + 2),
    )(functools.partial(_emb_body, batch, seq, s_per_w, rows_wave, d))

    out = run(ids_flat, token_table, pos_table)
    return out.reshape(batch, seq, d)
